# Initial kernel scaffold; baseline (speedup 1.0000x reference)
import jax
import jax.numpy as jnp
from jax.experimental import pallas as pl


def kernel(x_user, x_item, edge_index_ui, edge_index_iu, edge_label_index, W1_ui, b1_ui, W1_iu, b1_iu, W2_ui, b2_ui, W2_iu, b2_iu, Wd1, bd1, Wd2, bd2):
    raise NotImplementedError("write your pallas kernel here")


# speedup vs baseline: 3.2589x; 3.2589x over previous
import functools

import jax
import jax.numpy as jnp
from jax import lax
from jax.experimental import pallas as pl
from jax.experimental.pallas import tpu as pltpu
from jax.experimental.pallas import tpu_sc as plsc

N = 10000
NP = 10240
E = 160000
EP = 163840
L = 200000
LP = 200704
D_IN = 256
D_HID = 512
D_OUT = 256
NC = 2
NS = 16
NW = NC * NS

_mesh = lambda: plsc.VectorSubcoreMesh(
    core_axis_name="c", subcore_axis_name="s", num_cores=NC, num_subcores=NS)


def _deg_call(su, di, si, du, zeros_np):
  EPH = EP // 2
  PT = EPH // NS
  NCHUNK = PT // 128

  @functools.partial(
      pl.kernel,
      out_type=jax.ShapeDtypeStruct((4, NC, NS, NP), jnp.float32),
      mesh=_mesh(),
      scratch_types=[
          pltpu.VMEM((128,), jnp.int32),
          pltpu.VMEM((NP,), jnp.float32),
          pltpu.VMEM((NP,), jnp.float32),
          pltpu.VMEM((NP,), jnp.float32),
          pltpu.VMEM((NP,), jnp.float32),
      ],
      compiler_params=pltpu.CompilerParams(needs_layout_passes=False),
  )
  def k(su_h, di_h, si_h, du_h, zeros_hbm, out_h, idx_v, h0, h1, h2, h3):
    c = lax.axis_index("c")
    s = lax.axis_index("s")
    hists = [h0, h1, h2, h3]
    for h in hists:
      pltpu.sync_copy(zeros_hbm, h)
    ones16 = jnp.ones((16,), jnp.float32)
    srcs = [su_h, di_h, si_h, du_h]
    for j in range(4):
      def body(kk, _, j=j):
        base = c * EPH + s * PT + kk * 128
        pltpu.sync_copy(srcs[j].at[pl.ds(base, 128)], idx_v)
        for t in range(8):
          iv = idx_v[pl.ds(t * 16, 16)]
          plsc.addupdate_scatter(hists[j], [iv], ones16)
        return 0
      lax.fori_loop(0, NCHUNK, body, 0)
    for j in range(4):
      pltpu.sync_copy(hists[j], out_h.at[j, c, s])

  return k(su, di, si, du, zeros_np)


def _agg_call(xs_ui, xs_iu, su, di, si, du, zeros2_h):
  PT = EP // NS
  NCHUNK = PT // 128

  @functools.partial(
      pl.kernel,
      out_type=(jax.ShapeDtypeStruct((NC, NP, 128), jnp.float32),
                jax.ShapeDtypeStruct((NC, NP, 128), jnp.float32)),
      mesh=_mesh(),
      scratch_types=[
          pltpu.VMEM((128,), jnp.int32),
          pltpu.VMEM((128,), jnp.int32),
          pltpu.VMEM((128, 128), jnp.float32),
          pltpu.VMEM_SHARED((NP, 128), jnp.float32),
          pltpu.SemaphoreType.DMA,
      ],
  )
  def k(xs_ui_h, xs_iu_h, su_h, di_h, si_h, du_h, z2_h,
        out_item_h, out_user_h, idx_s, idx_d, rows_v, acc, sem):
    c = lax.axis_index("c")
    s = lax.axis_index("s")
    for (xs_h, src_h, dst_h, out_h) in (
        (xs_ui_h, su_h, di_h, out_item_h),
        (xs_iu_h, si_h, du_h, out_user_h),
    ):
      for z in range(5):
        pltpu.sync_copy(z2_h, acc.at[pl.ds(s * 640 + z * 128, 128)])
      plsc.subcore_barrier()

      def body(kk, _, xs_h=xs_h, src_h=src_h, dst_h=dst_h):
        base = s * PT + kk * 128
        pltpu.sync_copy(src_h.at[pl.ds(base, 128)], idx_s)
        pltpu.sync_copy(dst_h.at[pl.ds(base, 128)], idx_d)
        pltpu.async_copy(xs_h.at[c].at[idx_s], rows_v, sem).wait()
        pltpu.sync_copy(rows_v, acc.at[idx_d], add=True)
        return 0
      lax.fori_loop(0, NCHUNK, body, 0)
      plsc.subcore_barrier()
      pltpu.sync_copy(acc.at[pl.ds(s * 640, 640)],
                      out_h.at[c, pl.ds(s * 640, 640)])
      plsc.subcore_barrier()

  return k(xs_ui, xs_iu, su, di, si, du, zeros2_h)


def _decode_call(u_arr, i_arr, sidx, didx, w2c, bd2b):
  C = 64
  PT = LP // NW
  NCHUNK = PT // C

  @functools.partial(
      pl.kernel,
      out_type=jax.ShapeDtypeStruct((LP,), jnp.float32),
      mesh=_mesh(),
      compiler_params=pltpu.CompilerParams(needs_layout_passes=False),
      scratch_types=[
          pltpu.VMEM((C,), jnp.int32),
          pltpu.VMEM((C,), jnp.int32),
          pltpu.VMEM((C, 512), jnp.float32),
          pltpu.VMEM((C, 512), jnp.float32),
          pltpu.VMEM((C * 16,), jnp.float32),
          pltpu.VMEM((C,), jnp.float32),
          pltpu.VMEM((512,), jnp.float32),
          pltpu.VMEM((16,), jnp.float32),
          pltpu.SemaphoreType.DMA,
          pltpu.SemaphoreType.DMA,
      ],
  )
  def k(u_h, i_h, s_h, d_h, w_hbm, b_hbm, out_h,
        sv, dv, urows, irows, part_v, outv, w_v, b_v, sem_u, sem_i):
    c = lax.axis_index("c")
    s = lax.axis_index("s")
    wid = c * NS + s
    pltpu.sync_copy(w_hbm, w_v)
    pltpu.sync_copy(b_hbm, b_v)
    wvals = [w_v[pl.ds(j * 16, 16)] for j in range(32)]
    bvec = b_v[...]

    def chunk(kk, _):
      base = wid * PT + kk * C
      pltpu.sync_copy(s_h.at[pl.ds(base, C)], sv)
      pltpu.sync_copy(d_h.at[pl.ds(base, C)], dv)
      cu = pltpu.async_copy(u_h.at[sv], urows, sem_u)
      ci = pltpu.async_copy(i_h.at[dv], irows, sem_i)
      cu.wait()
      ci.wait()

      def edge(e, _):
        acc = jnp.zeros((16,), jnp.float32)
        for j in range(32):
          u = urows[e, pl.ds(j * 16, 16)]
          i = irows[e, pl.ds(j * 16, 16)]
          acc = acc + jnp.maximum(u + i, 0.0) * wvals[j]
        part_v[pl.ds(e * 16, 16)] = acc
        return 0
      lax.fori_loop(0, C, edge, 0)

      for g in range(C // 16):
        acc2 = bvec
        for l in range(16):
          gi = lax.iota(jnp.int32, 16) * 16 + (g * 256 + l)
          acc2 = acc2 + plsc.load_gather(part_v, [gi])
        outv[pl.ds(g * 16, 16)] = acc2
      pltpu.sync_copy(outv, out_h.at[pl.ds(base, C)])
      return 0
    lax.fori_loop(0, NCHUNK, chunk, 0)

  return k(u_arr, i_arr, sidx, didx, w2c, bd2b)


def _scale_call(deg128, xu, xi):
  R = 256
  grid = (NP // R,)

  def body(deg_ref, xu_ref, xi_ref, sc_ref, xsu_ref, xsi_ref):
    p = deg_ref[...]
    deg = jnp.stack(
        [jnp.sum(p[32 * j:32 * (j + 1), :], axis=0) for j in range(4)], axis=0)
    r = jnp.where(deg > 0.0, lax.rsqrt(jnp.maximum(deg, 1e-12)), 0.0)
    sc_ref[...] = jnp.concatenate([r, jnp.zeros_like(r)], axis=0)
    xsu_ref[...] = xu_ref[...] * r[0][:, None]
    xsi_ref[...] = xi_ref[...] * r[2][:, None]

  return pl.pallas_call(
      body,
      grid=grid,
      in_specs=[
          pl.BlockSpec((128, R), lambda i: (0, i)),
          pl.BlockSpec((R, D_IN), lambda i: (i, 0)),
          pl.BlockSpec((R, D_IN), lambda i: (i, 0)),
      ],
      out_specs=[
          pl.BlockSpec((8, R), lambda i: (0, i)),
          pl.BlockSpec((R, D_IN), lambda i: (i, 0)),
          pl.BlockSpec((R, D_IN), lambda i: (i, 0)),
      ],
      out_shape=[
          jax.ShapeDtypeStruct((8, NP), jnp.float32),
          jax.ShapeDtypeStruct((NP, D_IN), jnp.float32),
          jax.ShapeDtypeStruct((NP, D_IN), jnp.float32),
      ],
  )(deg128, xu, xi)


def _layers_call(agg_item, agg_user, scales, w1ui, b1ui8, w2iu,
                 w1iu, b1iu8, w2ui):
  R = 256
  grid = (NP // R,)

  def body(ai_ref, au_ref, sc_ref, w1ui_ref, b1ui_ref, w2iu_ref,
           w1iu_ref, b1iu_ref, w2ui_ref, msu_ref, msi_ref):
    r = sc_ref[...]
    a_i = ai_ref[...] * r[1][:, None]
    h_i = jnp.maximum(
        jnp.dot(a_i, w1ui_ref[...], preferred_element_type=jnp.float32)
        + b1ui_ref[0][None, :], 0.0)
    msi_ref[...] = jnp.dot(
        h_i, w2iu_ref[...], preferred_element_type=jnp.float32) * r[2][:, None]
    a_u = au_ref[...] * r[3][:, None]
    h_u = jnp.maximum(
        jnp.dot(a_u, w1iu_ref[...], preferred_element_type=jnp.float32)
        + b1iu_ref[0][None, :], 0.0)
    msu_ref[...] = jnp.dot(
        h_u, w2ui_ref[...], preferred_element_type=jnp.float32) * r[0][:, None]

  return pl.pallas_call(
      body,
      grid=grid,
      in_specs=[
          pl.BlockSpec((R, D_IN), lambda i: (i, 0)),
          pl.BlockSpec((R, D_IN), lambda i: (i, 0)),
          pl.BlockSpec((8, R), lambda i: (0, i)),
          pl.BlockSpec((D_IN, D_HID), lambda i: (0, 0)),
          pl.BlockSpec((8, D_HID), lambda i: (0, 0)),
          pl.BlockSpec((D_HID, D_OUT), lambda i: (0, 0)),
          pl.BlockSpec((D_IN, D_HID), lambda i: (0, 0)),
          pl.BlockSpec((8, D_HID), lambda i: (0, 0)),
          pl.BlockSpec((D_HID, D_OUT), lambda i: (0, 0)),
      ],
      out_specs=[
          pl.BlockSpec((R, D_OUT), lambda i: (i, 0)),
          pl.BlockSpec((R, D_OUT), lambda i: (i, 0)),
      ],
      out_shape=[
          jax.ShapeDtypeStruct((NP, D_OUT), jnp.float32),
          jax.ShapeDtypeStruct((NP, D_OUT), jnp.float32),
      ],
  )(agg_item, agg_user, scales, w1ui, b1ui8, w2iu, w1iu, b1iu8, w2ui)


def _uv_call(zagg_item, zagg_user, scales, b2ui8, b2iu8,
             wu_cat, bu_cat8, wi_cat, bi_cat8):
  R = 256
  grid = (NP // R,)

  def body(zi_ref, zu_ref, sc_ref, b2ui_ref, b2iu_ref,
           wu_ref, bu_ref, wi_ref, bi_ref, u_ref, i_ref):
    r = sc_ref[...]
    z_u = zu_ref[...] * r[3][:, None] + b2iu_ref[0][None, :]
    u_ref[...] = jnp.dot(
        z_u, wu_ref[...], preferred_element_type=jnp.float32) + bu_ref[0][None, :]
    z_i = zi_ref[...] * r[1][:, None] + b2ui_ref[0][None, :]
    i_ref[...] = jnp.dot(
        z_i, wi_ref[...], preferred_element_type=jnp.float32) + bi_ref[0][None, :]

  return pl.pallas_call(
      body,
      grid=grid,
      in_specs=[
          pl.BlockSpec((R, D_OUT), lambda i: (i, 0)),
          pl.BlockSpec((R, D_OUT), lambda i: (i, 0)),
          pl.BlockSpec((8, R), lambda i: (0, i)),
          pl.BlockSpec((8, D_OUT), lambda i: (0, 0)),
          pl.BlockSpec((8, D_OUT), lambda i: (0, 0)),
          pl.BlockSpec((D_OUT, 2 * D_OUT), lambda i: (0, 0)),
          pl.BlockSpec((8, 2 * D_OUT), lambda i: (0, 0)),
          pl.BlockSpec((D_OUT, 2 * D_OUT), lambda i: (0, 0)),
          pl.BlockSpec((8, 2 * D_OUT), lambda i: (0, 0)),
      ],
      out_specs=[
          pl.BlockSpec((R, 2 * D_OUT), lambda i: (i, 0)),
          pl.BlockSpec((R, 2 * D_OUT), lambda i: (i, 0)),
      ],
      out_shape=[
          jax.ShapeDtypeStruct((NP, 2 * D_OUT), jnp.float32),
          jax.ShapeDtypeStruct((NP, 2 * D_OUT), jnp.float32),
      ],
  )(zagg_item, zagg_user, scales, b2ui8, b2iu8, wu_cat, bu_cat8, wi_cat, bi_cat8)


def kernel(x_user, x_item, edge_index_ui, edge_index_iu, edge_label_index,
           W1_ui, b1_ui, W1_iu, b1_iu, W2_ui, b2_ui, W2_iu, b2_iu,
           Wd1, bd1, Wd2, bd2):
  i32 = jnp.int32
  pad_e = jnp.full((EP - E,), N, i32)
  su = jnp.concatenate([edge_index_ui[0].astype(i32), pad_e])
  di = jnp.concatenate([edge_index_ui[1].astype(i32), pad_e])
  si = jnp.concatenate([edge_index_iu[0].astype(i32), pad_e])
  du = jnp.concatenate([edge_index_iu[1].astype(i32), pad_e])
  pad_l = jnp.full((LP - L,), N, i32)
  ls = jnp.concatenate([edge_label_index[0].astype(i32), pad_l])
  ld = jnp.concatenate([edge_label_index[1].astype(i32), pad_l])

  zpadn = jnp.zeros((NP - N, D_IN), jnp.float32)
  xu = jnp.concatenate([x_user, zpadn], axis=0)
  xi = jnp.concatenate([x_item, zpadn], axis=0)

  zeros_np = jnp.zeros((NP,), jnp.float32)
  zeros2_h = jnp.zeros((128, 128), jnp.float32)

  deg_parts = _deg_call(su, di, si, du, zeros_np)
  deg128 = deg_parts.reshape(128, NP)
  scales, xsu, xsi = _scale_call(deg128, xu, xi)

  split = lambda a: a.reshape(NP, 2, 128).transpose(1, 0, 2)
  unsplit = lambda a: a.transpose(1, 0, 2).reshape(NP, 256)

  agg_item, agg_user = _agg_call(split(xsu), split(xsi), su, di, si, du, zeros2_h)
  tile8 = lambda b: jnp.tile(b[None, :], (8, 1))
  ms_user, ms_item = _layers_call(
      unsplit(agg_item), unsplit(agg_user), scales,
      W1_ui, tile8(b1_ui), W2_iu, W1_iu, tile8(b1_iu), W2_ui)

  zagg_item, zagg_user = _agg_call(
      split(ms_user), split(ms_item), su, di, si, du, zeros2_h)
  wt = Wd1[:D_OUT, :]
  wb = Wd1[D_OUT:, :]
  zz = jnp.zeros((D_OUT,), jnp.float32)
  u_arr, i_arr = _uv_call(
      unsplit(zagg_item), unsplit(zagg_user), scales,
      tile8(b2_ui), tile8(b2_iu),
      jnp.concatenate([wt, wb], axis=1), tile8(jnp.concatenate([bd1, zz])),
      jnp.concatenate([wb, wt], axis=1), tile8(jnp.concatenate([zz, bd1])))

  w2c = 0.5 * jnp.concatenate([Wd2[:, 0], Wd2[:, 0]])
  bd2b = jnp.broadcast_to(bd2, (16,)).astype(jnp.float32)
  outd = _decode_call(u_arr, i_arr, ls, ld, w2c, bd2b)

  out = outd[:L]
  return jnp.concatenate([-out, out])


# speedup vs baseline: 4.2572x; 1.3063x over previous
import functools

import jax
import jax.numpy as jnp
from jax import lax
from jax.experimental import pallas as pl
from jax.experimental.pallas import tpu as pltpu
from jax.experimental.pallas import tpu_sc as plsc

N = 10000
NP = 10240
E = 160000
EP = 163840
L = 200000
LP = 200704
D_IN = 256
D_HID = 512
D_OUT = 256
NC = 2
NS = 16
NW = NC * NS

_mesh = lambda: plsc.VectorSubcoreMesh(
    core_axis_name="c", subcore_axis_name="s", num_cores=NC, num_subcores=NS)


def _deg_call(su, di, si, du, zeros_np):
  EPH = EP // 2
  PT = EPH // NS
  NCHUNK = PT // 128

  @functools.partial(
      pl.kernel,
      out_type=jax.ShapeDtypeStruct((4, NC, NS, NP), jnp.float32),
      mesh=_mesh(),
      scratch_types=[
          pltpu.VMEM((128,), jnp.int32),
          pltpu.VMEM((NP,), jnp.float32),
          pltpu.VMEM((NP,), jnp.float32),
          pltpu.VMEM((NP,), jnp.float32),
          pltpu.VMEM((NP,), jnp.float32),
      ],
      compiler_params=pltpu.CompilerParams(needs_layout_passes=False),
  )
  def k(su_h, di_h, si_h, du_h, zeros_hbm, out_h, idx_v, h0, h1, h2, h3):
    c = lax.axis_index("c")
    s = lax.axis_index("s")
    hists = [h0, h1, h2, h3]
    for h in hists:
      pltpu.sync_copy(zeros_hbm, h)
    ones16 = jnp.ones((16,), jnp.float32)
    srcs = [su_h, di_h, si_h, du_h]
    for j in range(4):
      def body(kk, _, j=j):
        base = c * EPH + s * PT + kk * 128
        pltpu.sync_copy(srcs[j].at[pl.ds(base, 128)], idx_v)
        for t in range(8):
          iv = idx_v[pl.ds(t * 16, 16)]
          plsc.addupdate_scatter(hists[j], [iv], ones16)
        return 0
      lax.fori_loop(0, NCHUNK, body, 0)
    for j in range(4):
      pltpu.sync_copy(hists[j], out_h.at[j, c, s])

  return k(su, di, si, du, zeros_np)


def _agg_call(xs_ui, xs_iu, su, di, si, du, zeros2_h):
  PT = EP // NS
  NCHUNK = PT // 128

  @functools.partial(
      pl.kernel,
      out_type=(jax.ShapeDtypeStruct((NC, NP, 128), jnp.float32),
                jax.ShapeDtypeStruct((NC, NP, 128), jnp.float32)),
      mesh=_mesh(),
      scratch_types=[
          pltpu.VMEM((2, 128), jnp.int32),
          pltpu.VMEM((2, 128), jnp.int32),
          pltpu.VMEM((128, 128), jnp.float32),
          pltpu.VMEM((128, 128), jnp.float32),
          pltpu.VMEM_SHARED((NP, 128), jnp.float32),
          pltpu.SemaphoreType.DMA,
          pltpu.SemaphoreType.DMA,
      ],
  )
  def k(xs_ui_h, xs_iu_h, su_h, di_h, si_h, du_h, z2_h,
        out_item_h, out_user_h, idx_s, idx_d, rows_a, rows_b, acc,
        sem_a, sem_b):
    c = lax.axis_index("c")
    s = lax.axis_index("s")
    for (xs_h, src_h, dst_h, out_h) in (
        (xs_ui_h, su_h, di_h, out_item_h),
        (xs_iu_h, si_h, du_h, out_user_h),
    ):
      for z in range(5):
        pltpu.sync_copy(z2_h, acc.at[pl.ds(s * 640 + z * 128, 128)])
      plsc.subcore_barrier()

      def stage(kk, b, rows, sem, xs_h=xs_h, src_h=src_h, dst_h=dst_h):
        base = s * PT + kk * 128
        pltpu.sync_copy(src_h.at[pl.ds(base, 128)], idx_s.at[b])
        pltpu.sync_copy(dst_h.at[pl.ds(base, 128)], idx_d.at[b])
        pltpu.async_copy(xs_h.at[c].at[idx_s.at[b]], rows, sem)

      def drain(kk, b, rows, sem, xs_h=xs_h):
        pltpu.make_async_copy(xs_h.at[c].at[idx_s.at[b]], rows, sem).wait()
        pltpu.sync_copy(rows, acc.at[idx_d.at[b]], add=True)

      stage(0, 0, rows_a, sem_a)

      def body(kk, _):
        stage(2 * kk + 1, 1, rows_b, sem_b)
        drain(2 * kk, 0, rows_a, sem_a)

        @pl.when(kk < NCHUNK // 2 - 1)
        def _():
          stage(2 * kk + 2, 0, rows_a, sem_a)
        drain(2 * kk + 1, 1, rows_b, sem_b)
        return 0
      lax.fori_loop(0, NCHUNK // 2, body, 0)
      plsc.subcore_barrier()
      pltpu.sync_copy(acc.at[pl.ds(s * 640, 640)],
                      out_h.at[c, pl.ds(s * 640, 640)])
      plsc.subcore_barrier()

  return k(xs_ui, xs_iu, su, di, si, du, zeros2_h)


def _decode_call(u_arr, i_arr, sidx, didx, w2c, bd2b):
  C = 32
  PT = LP // NW
  NCHUNK = PT // C

  @functools.partial(
      pl.kernel,
      out_type=jax.ShapeDtypeStruct((LP,), jnp.float32),
      mesh=_mesh(),
      compiler_params=pltpu.CompilerParams(needs_layout_passes=False),
      scratch_types=[
          pltpu.VMEM((2, C), jnp.int32),
          pltpu.VMEM((2, C), jnp.int32),
          pltpu.VMEM((C, 512), jnp.float32),
          pltpu.VMEM((C, 512), jnp.float32),
          pltpu.VMEM((C, 512), jnp.float32),
          pltpu.VMEM((C, 512), jnp.float32),
          pltpu.VMEM((C * 16,), jnp.float32),
          pltpu.VMEM((C,), jnp.float32),
          pltpu.VMEM((512,), jnp.float32),
          pltpu.VMEM((16,), jnp.float32),
          pltpu.SemaphoreType.DMA,
          pltpu.SemaphoreType.DMA,
      ],
  )
  def k(u_h, i_h, s_h, d_h, w_hbm, b_hbm, out_h,
        sv, dv, ur_a, ir_a, ur_b, ir_b, part_v, outv, w_v, b_v,
        sem_a, sem_b):
    c = lax.axis_index("c")
    s = lax.axis_index("s")
    wid = c * NS + s
    pltpu.sync_copy(w_hbm, w_v)
    pltpu.sync_copy(b_hbm, b_v)
    wvals = [w_v[pl.ds(j * 16, 16)] for j in range(32)]
    bvec = b_v[...]

    def stage(kk, b, ur, ir, sem):
      base = wid * PT + kk * C
      pltpu.sync_copy(s_h.at[pl.ds(base, C)], sv.at[b])
      pltpu.sync_copy(d_h.at[pl.ds(base, C)], dv.at[b])
      pltpu.async_copy(u_h.at[sv.at[b]], ur, sem)
      pltpu.async_copy(i_h.at[dv.at[b]], ir, sem)

    def compute(kk, b, ur, ir, sem):
      base = wid * PT + kk * C
      pltpu.make_async_copy(u_h.at[sv.at[b]], ur, sem).wait()
      pltpu.make_async_copy(i_h.at[dv.at[b]], ir, sem).wait()

      def edge(e, _, ur=ur, ir=ir):
        acc = jnp.zeros((16,), jnp.float32)
        for j in range(32):
          u = ur[e, pl.ds(j * 16, 16)]
          i = ir[e, pl.ds(j * 16, 16)]
          acc = acc + jnp.maximum(u + i, 0.0) * wvals[j]
        part_v[pl.ds(e * 16, 16)] = acc
        return 0
      lax.fori_loop(0, C, edge, 0)

      for g in range(C // 16):
        acc2 = bvec
        for l in range(16):
          gi = lax.iota(jnp.int32, 16) * 16 + (g * 256 + l)
          acc2 = acc2 + plsc.load_gather(part_v, [gi])
        outv[pl.ds(g * 16, 16)] = acc2
      pltpu.sync_copy(outv, out_h.at[pl.ds(base, C)])

    stage(0, 0, ur_a, ir_a, sem_a)

    def body(kk, _):
      stage(2 * kk + 1, 1, ur_b, ir_b, sem_b)
      compute(2 * kk, 0, ur_a, ir_a, sem_a)

      @pl.when(kk < NCHUNK // 2 - 1)
      def _():
        stage(2 * kk + 2, 0, ur_a, ir_a, sem_a)
      compute(2 * kk + 1, 1, ur_b, ir_b, sem_b)
      return 0
    lax.fori_loop(0, NCHUNK // 2, body, 0)

  return k(u_arr, i_arr, sidx, didx, w2c, bd2b)


def _scale_call(deg128, xu, xi):
  R = 256
  grid = (NP // R,)

  def body(deg_ref, xu_ref, xi_ref, sc_ref, xsu_ref, xsi_ref):
    p = deg_ref[...]
    deg = jnp.stack(
        [jnp.sum(p[32 * j:32 * (j + 1), :], axis=0) for j in range(4)], axis=0)
    r = jnp.where(deg > 0.0, lax.rsqrt(jnp.maximum(deg, 1e-12)), 0.0)
    sc_ref[...] = jnp.concatenate([r, jnp.zeros_like(r)], axis=0)
    xsu_ref[...] = xu_ref[...] * r[0][:, None]
    xsi_ref[...] = xi_ref[...] * r[2][:, None]

  return pl.pallas_call(
      body,
      grid=grid,
      in_specs=[
          pl.BlockSpec((128, R), lambda i: (0, i)),
          pl.BlockSpec((R, D_IN), lambda i: (i, 0)),
          pl.BlockSpec((R, D_IN), lambda i: (i, 0)),
      ],
      out_specs=[
          pl.BlockSpec((8, R), lambda i: (0, i)),
          pl.BlockSpec((R, D_IN), lambda i: (i, 0)),
          pl.BlockSpec((R, D_IN), lambda i: (i, 0)),
      ],
      out_shape=[
          jax.ShapeDtypeStruct((8, NP), jnp.float32),
          jax.ShapeDtypeStruct((NP, D_IN), jnp.float32),
          jax.ShapeDtypeStruct((NP, D_IN), jnp.float32),
      ],
  )(deg128, xu, xi)


def _layers_call(agg_item, agg_user, scales, w1ui, b1ui8, w2iu,
                 w1iu, b1iu8, w2ui):
  R = 256
  grid = (NP // R,)

  def body(ai_ref, au_ref, sc_ref, w1ui_ref, b1ui_ref, w2iu_ref,
           w1iu_ref, b1iu_ref, w2ui_ref, msu_ref, msi_ref):
    r = sc_ref[...]
    a_i = ai_ref[...] * r[1][:, None]
    h_i = jnp.maximum(
        jnp.dot(a_i, w1ui_ref[...], preferred_element_type=jnp.float32)
        + b1ui_ref[0][None, :], 0.0)
    msi_ref[...] = jnp.dot(
        h_i, w2iu_ref[...], preferred_element_type=jnp.float32) * r[2][:, None]
    a_u = au_ref[...] * r[3][:, None]
    h_u = jnp.maximum(
        jnp.dot(a_u, w1iu_ref[...], preferred_element_type=jnp.float32)
        + b1iu_ref[0][None, :], 0.0)
    msu_ref[...] = jnp.dot(
        h_u, w2ui_ref[...], preferred_element_type=jnp.float32) * r[0][:, None]

  return pl.pallas_call(
      body,
      grid=grid,
      in_specs=[
          pl.BlockSpec((R, D_IN), lambda i: (i, 0)),
          pl.BlockSpec((R, D_IN), lambda i: (i, 0)),
          pl.BlockSpec((8, R), lambda i: (0, i)),
          pl.BlockSpec((D_IN, D_HID), lambda i: (0, 0)),
          pl.BlockSpec((8, D_HID), lambda i: (0, 0)),
          pl.BlockSpec((D_HID, D_OUT), lambda i: (0, 0)),
          pl.BlockSpec((D_IN, D_HID), lambda i: (0, 0)),
          pl.BlockSpec((8, D_HID), lambda i: (0, 0)),
          pl.BlockSpec((D_HID, D_OUT), lambda i: (0, 0)),
      ],
      out_specs=[
          pl.BlockSpec((R, D_OUT), lambda i: (i, 0)),
          pl.BlockSpec((R, D_OUT), lambda i: (i, 0)),
      ],
      out_shape=[
          jax.ShapeDtypeStruct((NP, D_OUT), jnp.float32),
          jax.ShapeDtypeStruct((NP, D_OUT), jnp.float32),
      ],
  )(agg_item, agg_user, scales, w1ui, b1ui8, w2iu, w1iu, b1iu8, w2ui)


def _uv_call(zagg_item, zagg_user, scales, b2ui8, b2iu8,
             wu_cat, bu_cat8, wi_cat, bi_cat8):
  R = 256
  grid = (NP // R,)

  def body(zi_ref, zu_ref, sc_ref, b2ui_ref, b2iu_ref,
           wu_ref, bu_ref, wi_ref, bi_ref, u_ref, i_ref):
    r = sc_ref[...]
    z_u = zu_ref[...] * r[3][:, None] + b2iu_ref[0][None, :]
    u_ref[...] = jnp.dot(
        z_u, wu_ref[...], preferred_element_type=jnp.float32) + bu_ref[0][None, :]
    z_i = zi_ref[...] * r[1][:, None] + b2ui_ref[0][None, :]
    i_ref[...] = jnp.dot(
        z_i, wi_ref[...], preferred_element_type=jnp.float32) + bi_ref[0][None, :]

  return pl.pallas_call(
      body,
      grid=grid,
      in_specs=[
          pl.BlockSpec((R, D_OUT), lambda i: (i, 0)),
          pl.BlockSpec((R, D_OUT), lambda i: (i, 0)),
          pl.BlockSpec((8, R), lambda i: (0, i)),
          pl.BlockSpec((8, D_OUT), lambda i: (0, 0)),
          pl.BlockSpec((8, D_OUT), lambda i: (0, 0)),
          pl.BlockSpec((D_OUT, 2 * D_OUT), lambda i: (0, 0)),
          pl.BlockSpec((8, 2 * D_OUT), lambda i: (0, 0)),
          pl.BlockSpec((D_OUT, 2 * D_OUT), lambda i: (0, 0)),
          pl.BlockSpec((8, 2 * D_OUT), lambda i: (0, 0)),
      ],
      out_specs=[
          pl.BlockSpec((R, 2 * D_OUT), lambda i: (i, 0)),
          pl.BlockSpec((R, 2 * D_OUT), lambda i: (i, 0)),
      ],
      out_shape=[
          jax.ShapeDtypeStruct((NP, 2 * D_OUT), jnp.float32),
          jax.ShapeDtypeStruct((NP, 2 * D_OUT), jnp.float32),
      ],
  )(zagg_item, zagg_user, scales, b2ui8, b2iu8, wu_cat, bu_cat8, wi_cat, bi_cat8)


def kernel(x_user, x_item, edge_index_ui, edge_index_iu, edge_label_index,
           W1_ui, b1_ui, W1_iu, b1_iu, W2_ui, b2_ui, W2_iu, b2_iu,
           Wd1, bd1, Wd2, bd2):
  i32 = jnp.int32
  pad_e = jnp.full((EP - E,), N, i32)
  su = jnp.concatenate([edge_index_ui[0].astype(i32), pad_e])
  di = jnp.concatenate([edge_index_ui[1].astype(i32), pad_e])
  si = jnp.concatenate([edge_index_iu[0].astype(i32), pad_e])
  du = jnp.concatenate([edge_index_iu[1].astype(i32), pad_e])
  pad_l = jnp.full((LP - L,), N, i32)
  ls = jnp.concatenate([edge_label_index[0].astype(i32), pad_l])
  ld = jnp.concatenate([edge_label_index[1].astype(i32), pad_l])

  zpadn = jnp.zeros((NP - N, D_IN), jnp.float32)
  xu = jnp.concatenate([x_user, zpadn], axis=0)
  xi = jnp.concatenate([x_item, zpadn], axis=0)

  zeros_np = jnp.zeros((NP,), jnp.float32)
  zeros2_h = jnp.zeros((128, 128), jnp.float32)

  deg_parts = _deg_call(su, di, si, du, zeros_np)
  deg128 = deg_parts.reshape(128, NP)
  scales, xsu, xsi = _scale_call(deg128, xu, xi)

  split = lambda a: a.reshape(NP, 2, 128).transpose(1, 0, 2)
  unsplit = lambda a: a.transpose(1, 0, 2).reshape(NP, 256)

  agg_item, agg_user = _agg_call(split(xsu), split(xsi), su, di, si, du, zeros2_h)
  tile8 = lambda b: jnp.tile(b[None, :], (8, 1))
  ms_user, ms_item = _layers_call(
      unsplit(agg_item), unsplit(agg_user), scales,
      W1_ui, tile8(b1_ui), W2_iu, W1_iu, tile8(b1_iu), W2_ui)

  zagg_item, zagg_user = _agg_call(
      split(ms_user), split(ms_item), su, di, si, du, zeros2_h)
  wt = Wd1[:D_OUT, :]
  wb = Wd1[D_OUT:, :]
  zz = jnp.zeros((D_OUT,), jnp.float32)
  u_arr, i_arr = _uv_call(
      unsplit(zagg_item), unsplit(zagg_user), scales,
      tile8(b2_ui), tile8(b2_iu),
      jnp.concatenate([wt, wb], axis=1), tile8(jnp.concatenate([bd1, zz])),
      jnp.concatenate([wb, wt], axis=1), tile8(jnp.concatenate([zz, bd1])))

  w2c = 0.5 * jnp.concatenate([Wd2[:, 0], Wd2[:, 0]])
  bd2b = jnp.broadcast_to(bd2, (16,)).astype(jnp.float32)
  outd = _decode_call(u_arr, i_arr, ls, ld, w2c, bd2b)

  out = outd[:L]
  return jnp.concatenate([-out, out])
